# baseline (speedup 1.0000x reference)
import jax
import jax.numpy as jnp
from jax.experimental import pallas as pl
from jax.experimental.pallas import tpu as pltpu

_STRIDE = 8.0
_NG = 76
_NP = _NG * _NG
_NCH = 85


def _decode_kernel(size_ref, x_ref, o_ref):
    img = size_ref[0, 0]
    x = x_ref[0]

    e = jnp.clip(jnp.exp(x[0:4, :]) * _STRIDE, 0.0, img)
    l_ = e[0:1, :]
    t_ = e[1:2, :]
    r_ = e[2:3, :]
    b_ = e[3:4, :]

    pos = jax.lax.broadcasted_iota(jnp.int32, (1, _NP), 1)
    cx = (pos % _NG).astype(jnp.float32) * _STRIDE + (_STRIDE / 2.0)
    cy = (pos // _NG).astype(jnp.float32) * _STRIDE + (_STRIDE / 2.0)

    bx = cx - (l_ - r_) * 0.5
    by = cy - (t_ - b_) * 0.5
    bw = l_ + r_
    bh = t_ + b_

    rest = jax.nn.sigmoid(x[4:, :])
    y = jnp.concatenate([bx, by, bw, bh, rest], axis=0)
    yt = y.T.reshape(_NP // 4, 4, _NCH)
    v = jnp.concatenate([yt[:, m, :] for m in range(4)], axis=1)
    o_ref[0] = v


def kernel(raw, img_size):
    nB = raw.shape[0]
    x = raw.reshape(nB, _NCH, _NP)
    size = jnp.asarray(img_size, jnp.float32).reshape(1, 1)
    out = pl.pallas_call(
        _decode_kernel,
        grid=(nB,),
        in_specs=[
            pl.BlockSpec(memory_space=pltpu.SMEM),
            pl.BlockSpec((1, _NCH, _NP), lambda b: (b, 0, 0)),
        ],
        out_specs=pl.BlockSpec((1, _NP // 4, 4 * _NCH), lambda b: (b, 0, 0)),
        out_shape=jax.ShapeDtypeStruct((nB, _NP // 4, 4 * _NCH), jnp.float32),
    )(size, x)
    return out.reshape(nB, _NP, _NCH)
